# Initial kernel scaffold; baseline (speedup 1.0000x reference)
import jax
import jax.numpy as jnp
from jax.experimental import pallas as pl


def kernel(ids, embed_table, W1, b1, W2, b2):
    raise NotImplementedError("write your pallas kernel here")


# speedup vs baseline: 4.8508x; 4.8508x over previous
import functools

import jax
import jax.numpy as jnp
from jax import lax
from jax.experimental import pallas as pl
from jax.experimental.pallas import tpu as pltpu
from jax.experimental.pallas import tpu_sc as plsc


def _mlp_table_body(tab_ref, w1_ref, b1_ref, w2_ref, b2_ref, y_ref):
    h = jnp.dot(tab_ref[...], w1_ref[...], preferred_element_type=jnp.float32)
    h = jnp.maximum(h + b1_ref[...], 0.0)
    y_ref[...] = (
        jnp.dot(h, w2_ref[...], preferred_element_type=jnp.float32) + b2_ref[...]
    )


def _compute_vocab_outputs(embed_table, W1, b1, W2, b2):
    V = embed_table.shape[0]
    H = W1.shape[1]
    O = W2.shape[1]
    return pl.pallas_call(
        _mlp_table_body,
        out_shape=jax.ShapeDtypeStruct((V, O), jnp.float32),
    )(embed_table, W1, b1.reshape(1, H), W2, b2.reshape(1, O))


@functools.cache
def _make_gather(V, D, N):
    info = plsc.get_sparse_core_info()
    NC, NS = info.num_cores, info.num_subcores
    NW = NC * NS
    assert N % NW == 0
    n_per = N // NW
    C = 256
    assert n_per % C == 0
    n_chunks = n_per // C

    mesh = plsc.VectorSubcoreMesh(core_axis_name="c", subcore_axis_name="s")

    @functools.partial(
        pl.kernel,
        out_type=jax.ShapeDtypeStruct((N, D), jnp.float32),
        mesh=mesh,
        scratch_types=[
            pltpu.VMEM((n_per,), jnp.int32),
            pltpu.VMEM((C, D), jnp.float32),
            pltpu.SemaphoreType.DMA,
        ],
    )
    def gather(y_hbm, idx_hbm, out_hbm, idx_v, rows_v, sem):
        wid = lax.axis_index("s") * NC + lax.axis_index("c")
        base = wid * n_per
        pltpu.sync_copy(idx_hbm.at[pl.ds(base, n_per)], idx_v)

        def body(g, carry):
            start = g * C
            pltpu.async_copy(
                y_hbm.at[idx_v.at[pl.ds(start, C)]], rows_v, sem
            ).wait()
            pltpu.sync_copy(rows_v, out_hbm.at[pl.ds(base + start, C)])
            return carry

        lax.fori_loop(0, n_chunks, body, 0)

    return gather


def kernel(ids, embed_table, W1, b1, W2, b2):
    B, L = ids.shape
    V = embed_table.shape[0]
    O = W2.shape[1]
    y = _compute_vocab_outputs(embed_table, W1, b1, W2, b2)
    idx = ids.reshape(-1).astype(jnp.int32)
    out = _make_gather(V, O, B * L)(y, idx)
    return out.reshape(B, L, O)


# speedup vs baseline: 5.0711x; 1.0454x over previous
import functools

import jax
import jax.numpy as jnp
from jax import lax
from jax.experimental import pallas as pl
from jax.experimental.pallas import tpu as pltpu
from jax.experimental.pallas import tpu_sc as plsc


def _mlp_table_body(tab_ref, w1_ref, b1_ref, w2_ref, b2_ref, y_ref):
    h = jnp.dot(tab_ref[...], w1_ref[...], preferred_element_type=jnp.float32)
    h = jnp.maximum(h + b1_ref[...], 0.0)
    y_ref[...] = (
        jnp.dot(h, w2_ref[...], preferred_element_type=jnp.float32) + b2_ref[...]
    )


def _compute_vocab_outputs(embed_table, W1, b1, W2, b2):
    V = embed_table.shape[0]
    H = W1.shape[1]
    O = W2.shape[1]
    return pl.pallas_call(
        _mlp_table_body,
        out_shape=jax.ShapeDtypeStruct((V, O), jnp.float32),
    )(embed_table, W1, b1.reshape(1, H), W2, b2.reshape(1, O))


@functools.cache
def _make_gather(V, D, N):
    info = plsc.get_sparse_core_info()
    NC, NS = info.num_cores, info.num_subcores
    NW = NC * NS
    assert N % NW == 0
    n_per = N // NW
    C = 200
    assert n_per % (2 * C) == 0
    n_chunks = n_per // C

    mesh = plsc.VectorSubcoreMesh(core_axis_name="c", subcore_axis_name="s")

    @functools.partial(
        pl.kernel,
        out_type=jax.ShapeDtypeStruct((N, D), jnp.float32),
        mesh=mesh,
        scratch_types=[
            pltpu.VMEM((n_per,), jnp.int32),
            pltpu.VMEM((C, D), jnp.float32),
            pltpu.VMEM((C, D), jnp.float32),
            pltpu.SemaphoreType.DMA,
            pltpu.SemaphoreType.DMA,
        ],
    )
    def gather(y_hbm, idx_hbm, out_hbm, idx_v, rows0, rows1, sem0, sem1):
        wid = lax.axis_index("s") * NC + lax.axis_index("c")
        base = wid * n_per
        pltpu.sync_copy(idx_hbm.at[pl.ds(base, n_per)], idx_v)

        def start_gather(g, buf, sem):
            pltpu.async_copy(y_hbm.at[idx_v.at[pl.ds(g * C, C)]], buf, sem)

        def wait_gather(buf, sem):
            pltpu.make_async_copy(y_hbm.at[idx_v.at[pl.ds(0, C)]], buf, sem).wait()

        start_gather(0, rows0, sem0)
        start_gather(1, rows1, sem1)

        def body(i, carry):
            g0 = 2 * i

            wait_gather(rows0, sem0)
            pltpu.sync_copy(rows0, out_hbm.at[pl.ds(base + g0 * C, C)])

            @pl.when(g0 + 2 < n_chunks)
            def _():
                start_gather(g0 + 2, rows0, sem0)

            wait_gather(rows1, sem1)
            pltpu.sync_copy(rows1, out_hbm.at[pl.ds(base + (g0 + 1) * C, C)])

            @pl.when(g0 + 3 < n_chunks)
            def _():
                start_gather(g0 + 3, rows1, sem1)

            return carry

        lax.fori_loop(0, n_chunks // 2, body, 0)

    return gather


def kernel(ids, embed_table, W1, b1, W2, b2):
    B, L = ids.shape
    V = embed_table.shape[0]
    O = W2.shape[1]
    y = _compute_vocab_outputs(embed_table, W1, b1, W2, b2)
    idx = ids.reshape(-1).astype(jnp.int32)
    out = _make_gather(V, O, B * L)(y, idx)
    return out.reshape(B, L, O)
